# Initial kernel scaffold; baseline (speedup 1.0000x reference)
import jax
import jax.numpy as jnp
from jax.experimental import pallas as pl


def kernel(input_tensor, grid):
    raise NotImplementedError("write your pallas kernel here")


# speedup vs baseline: 5.8579x; 5.8579x over previous
import functools

import jax
import jax.numpy as jnp
from jax import lax
from jax.experimental import pallas as pl
from jax.experimental.pallas import tpu as pltpu
from jax.experimental.pallas import tpu_sc as plsc

C = 128
H = 128
W = 128
HG = 7
WG = 25281
P = HG * WG

NC = 2
NS = 16
NW = NC * NS
LANES = 16

K = 64
NCH = 87
B_PER_W = K * NCH
P_PAD = B_PER_W * NW


_GATHER_DNUMS = lax.GatherDimensionNumbers(
    offset_dims=(), collapsed_slice_dims=(0,), start_index_map=(0,))


def _lane_bcast(v, j):
    idx = jnp.full((LANES, 1), j, dtype=jnp.int32)
    return lax.gather(v, idx, _GATHER_DNUMS, (1,),
                      mode=lax.GatherScatterMode.PROMISE_IN_BOUNDS)


def _axis_coords(g16, extent):
    x = (g16 + 1.0) * (extent * 0.5) - 0.5
    t = x.astype(jnp.int32)
    tf = t.astype(jnp.float32)
    neg = tf > x
    x0f = jnp.where(neg, tf - 1.0, tf)
    x0i = jnp.where(neg, t - 1, t)
    w1 = x - x0f
    w0 = 1.0 - w1
    lim = float(extent - 1)
    in0 = (x0f >= 0.0) & (x0f <= lim)
    in1 = (x0f >= -1.0) & (x0f <= lim - 1.0)
    return x0i, w0, w1, in0, in1


def _sc_grid_sample(table, gx, gy):
    mesh = plsc.VectorSubcoreMesh(core_axis_name="c", subcore_axis_name="s")

    @functools.partial(
        pl.kernel,
        out_type=jax.ShapeDtypeStruct((P_PAD, C), jnp.float32),
        mesh=mesh,
        scratch_types=[
            pltpu.VMEM((B_PER_W,), jnp.float32),
            pltpu.VMEM((B_PER_W,), jnp.float32),
            pltpu.VMEM((4, K), jnp.int32),
            pltpu.VMEM((4, K), jnp.float32),
            pltpu.VMEM((4, K, C), jnp.float32),
            pltpu.VMEM((K, C), jnp.float32),
            pltpu.SemaphoreType.DMA,
        ],
    )
    def body(table_hbm, gx_hbm, gy_hbm, out_hbm,
             gx_v, gy_v, idx_v, w_v, rows_v, out_v, sem):
        wid = lax.axis_index("s") * NC + lax.axis_index("c")
        base = wid * B_PER_W
        pltpu.sync_copy(gx_hbm.at[pl.ds(base, B_PER_W)], gx_v)
        pltpu.sync_copy(gy_hbm.at[pl.ds(base, B_PER_W)], gy_v)

        def chunk_body(ci, _):
            coff = ci * K
            for j in range(K // LANES):
                gx16 = gx_v[pl.ds(coff + j * LANES, LANES)]
                gy16 = gy_v[pl.ds(coff + j * LANES, LANES)]
                x0i, wx0, wx1, inx0, inx1 = _axis_coords(gx16, W)
                y0i, wy0, wy1, iny0, iny1 = _axis_coords(gy16, H)
                zero = jnp.zeros((LANES,), jnp.float32)
                xc0 = jnp.minimum(jnp.maximum(x0i, 0), W - 1)
                xc1 = jnp.minimum(jnp.maximum(x0i + 1, 0), W - 1)
                yb0 = jnp.minimum(jnp.maximum(y0i, 0), H - 1) * W
                yb1 = jnp.minimum(jnp.maximum(y0i + 1, 0), H - 1) * W
                sl = pl.ds(j * LANES, LANES)
                idx_v[0, sl] = yb0 + xc0
                idx_v[1, sl] = yb0 + xc1
                idx_v[2, sl] = yb1 + xc0
                idx_v[3, sl] = yb1 + xc1
                w_v[0, sl] = jnp.where(inx0 & iny0, wx0 * wy0, zero)
                w_v[1, sl] = jnp.where(inx1 & iny0, wx1 * wy0, zero)
                w_v[2, sl] = jnp.where(inx0 & iny1, wx0 * wy1, zero)
                w_v[3, sl] = jnp.where(inx1 & iny1, wx1 * wy1, zero)

            copies = [
                pltpu.async_copy(table_hbm.at[idx_v.at[t]], rows_v.at[t], sem)
                for t in range(4)
            ]
            for cp in copies:
                cp.wait()

            def group_body(g, _):
                p0 = g * LANES
                wt = [w_v[t, pl.ds(p0, LANES)] for t in range(4)]
                for j in range(LANES):
                    p = p0 + j
                    wb = [_lane_bcast(wt[t], j) for t in range(4)]
                    for cb in range(C // LANES):
                        cs = pl.ds(cb * LANES, LANES)
                        acc = rows_v[0, p, cs] * wb[0]
                        acc = acc + rows_v[1, p, cs] * wb[1]
                        acc = acc + rows_v[2, p, cs] * wb[2]
                        acc = acc + rows_v[3, p, cs] * wb[3]
                        out_v[p, cs] = acc
                return 0

            lax.fori_loop(0, K // LANES, group_body, 0, unroll=False)
            pltpu.sync_copy(out_v, out_hbm.at[pl.ds(base + coff, K)])
            return 0

        lax.fori_loop(0, NCH, chunk_body, 0, unroll=False)

    return body(table, gx, gy)


def kernel(input_tensor, grid):
    table = jnp.transpose(input_tensor[0], (1, 2, 0)).reshape(H * W, C)
    g = grid.reshape(P, 2)
    pad = jnp.full((P_PAD - P,), -2.0, dtype=jnp.float32)
    gx = jnp.concatenate([g[:, 0], pad])
    gy = jnp.concatenate([g[:, 1], pad])
    out_pc = _sc_grid_sample(table, gx, gy)
    return jnp.transpose(out_pc[:P]).reshape(1, C, HG, WG)
